# Initial kernel scaffold; baseline (speedup 1.0000x reference)
import jax
import jax.numpy as jnp
from jax.experimental import pallas as pl


def kernel(edge_index, user_features, product_features, Wu, bu, Wp, bp, W1, b1, g1, beta1, rm1, rv1, W2, b2, g2, beta2, rm2, rv2, Wpred, bpred):
    raise NotImplementedError("write your pallas kernel here")


# speedup vs baseline: 25.5165x; 25.5165x over previous
import functools

import jax
import jax.numpy as jnp
from jax import lax
from jax.experimental import pallas as pl
from jax.experimental.pallas import tpu as pltpu
from jax.experimental.pallas import tpu_sc as plsc

NU = 5000
NP = 5000
NPAD = 5120
NN = NU + NP
EMB = 256
E = 320000
NCELL = NU * NPAD
BN_EPS = 1e-5

R_RANGE = 1_600_000
DUMP = 2048
ACCN = R_RANGE + DUMP
NSWEEP = NCELL // (2 * R_RANGE)
EPT = E // 16
CCH = 2048
NCH = -(-EPT // CCH)
CLAST = EPT - (NCH - 1) * CCH
CROWS = CCH // 128
STRIPE = R_RANGE // 16
DCH = 4000
DRN = 10000

EPW = E // 32
NROW_C = -(-EPW // 128)
EPW_PAD = NROW_C * 128


def _cnt_sc(src, dst):
  mesh = plsc.VectorSubcoreMesh(core_axis_name="c", subcore_axis_name="s")

  @functools.partial(
      pl.kernel,
      out_type=jax.ShapeDtypeStruct((NCELL,), jnp.float32),
      mesh=mesh,
      scratch_types=[
          pltpu.VMEM((CCH,), jnp.int32),
          pltpu.VMEM((CCH,), jnp.int32),
          pltpu.VMEM((CROWS, 128), jnp.int32),
          pltpu.VMEM((128,), jnp.float32),
          pltpu.VMEM((DCH,), jnp.float32),
          pltpu.VMEM((DRN,), jnp.float32),
          pltpu.VMEM_SHARED((ACCN,), jnp.float32),
          pltpu.SemaphoreType.DMA,
      ],
  )
  def cnt_kernel(src_hbm, dst_hbm, cnt_hbm, src_v, dst_v, idx_v, ones_v,
                 zero_v, drain_v, acc_sh, sem):
    cid = lax.axis_index("c")
    sid = lax.axis_index("s")
    ebase = sid * EPT

    def fill_ones(i, carry):
      ones_v[pl.ds(i * 16, 16)] = jnp.full((16,), 1.0, jnp.float32)
      return carry

    lax.fori_loop(0, 8, fill_ones, 0)

    def fill_zero(i, carry):
      zero_v[pl.ds(i * 16, 16)] = jnp.full((16,), 0.0, jnp.float32)
      return carry

    lax.fori_loop(0, DCH // 16, fill_zero, 0)

    for sweep in range(NSWEEP):
      lo = (2 * sweep + cid) * R_RANGE

      def zero_stripe(k, carry):
        pltpu.sync_copy(zero_v, acc_sh.at[pl.ds(sid * STRIPE + k * DCH, DCH)])
        return carry

      lax.fori_loop(0, STRIPE // DCH, zero_stripe, 0)
      plsc.subcore_barrier()

      def do_chunk(c, carry):
        nload = jnp.where(c == NCH - 1, CLAST, CCH)

        @pl.when(c < NCH - 1)
        def _ld_full():
          pltpu.sync_copy(src_hbm.at[pl.ds(ebase + c * CCH, CCH)], src_v)
          pltpu.sync_copy(dst_hbm.at[pl.ds(ebase + c * CCH, CCH)], dst_v)

        @pl.when(c == NCH - 1)
        def _ld_tail():
          pltpu.sync_copy(src_hbm.at[pl.ds(ebase + c * CCH, CLAST)],
                          src_v.at[pl.ds(0, CLAST)])
          pltpu.sync_copy(dst_hbm.at[pl.ds(ebase + c * CCH, CLAST)],
                          dst_v.at[pl.ds(0, CLAST)])

        def build_idx(i, carry2):
          s16 = src_v[pl.ds(i * 16, 16)]
          d16 = dst_v[pl.ds(i * 16, 16)]
          flat = s16 * NPAD + d16 - NU
          local = flat - lo
          lane = i * 16 + lax.iota(jnp.int32, 16)
          ok = (lane < nload) & (local >= 0) & (local < R_RANGE)
          spill = R_RANGE + (flat & (DUMP - 1))
          idx_v[i // 8, pl.ds((i % 8) * 16, 16)] = jnp.where(ok, local, spill)
          return carry2

        lax.fori_loop(0, CCH // 16, build_idx, 0)

        handles = [
            pltpu.async_copy(ones_v, acc_sh.at[idx_v.at[j]], sem, add=True)
            for j in range(CROWS)
        ]
        for h in handles:
          h.wait()
        return carry

      lax.fori_loop(0, NCH, do_chunk, 0)
      plsc.subcore_barrier()

      def drain_chunk(k, carry):
        off = sid * STRIPE + k * DRN
        pltpu.sync_copy(acc_sh.at[pl.ds(off, DRN)], drain_v)
        pltpu.sync_copy(drain_v, cnt_hbm.at[pl.ds(lo + off, DRN)])
        return carry

      lax.fori_loop(0, STRIPE // DRN, drain_chunk, 0)

  return cnt_kernel(src, dst)


def _gather_sc(src, dst, mflat):
  mesh = plsc.VectorSubcoreMesh(core_axis_name="c", subcore_axis_name="s")

  @functools.partial(
      pl.kernel,
      out_type=jax.ShapeDtypeStruct((E,), jnp.float32),
      mesh=mesh,
      scratch_types=[
          pltpu.VMEM((EPW_PAD,), jnp.int32),
          pltpu.VMEM((EPW_PAD,), jnp.int32),
          pltpu.VMEM((NROW_C, 128), jnp.int32),
          pltpu.VMEM((EPW_PAD,), jnp.float32),
          pltpu.SemaphoreType.DMA,
      ],
  )
  def gather_kernel(src_hbm, dst_hbm, m_hbm, out_hbm, src_v, dst_v, idx_v,
                    gbuf_v, sem):
    cid = lax.axis_index("c")
    sid = lax.axis_index("s")
    w = sid * 2 + cid
    ebase = w * EPW
    pltpu.sync_copy(src_hbm.at[pl.ds(ebase, EPW)], src_v.at[pl.ds(0, EPW)])
    pltpu.sync_copy(dst_hbm.at[pl.ds(ebase, EPW)], dst_v.at[pl.ds(0, EPW)])

    def fill_pad(i, carry):
      src_v[pl.ds(EPW + i * 16, 16)] = jnp.zeros((16,), jnp.int32)
      dst_v[pl.ds(EPW + i * 16, 16)] = NU + lax.iota(jnp.int32, 16)
      return carry

    lax.fori_loop(0, (EPW_PAD - EPW) // 16, fill_pad, 0)

    def build_idx(i, carry):
      s16 = src_v[pl.ds(i * 16, 16)]
      d16 = dst_v[pl.ds(i * 16, 16)]
      idx_v[i // 8, pl.ds((i % 8) * 16, 16)] = s16 * NPAD + d16 - NU
      return carry

    lax.fori_loop(0, NROW_C * 8, build_idx, 0)

    handles = [
        pltpu.async_copy(m_hbm.at[idx_v.at[j]],
                         gbuf_v.at[pl.ds(j * 128, 128)], sem)
        for j in range(NROW_C)
    ]
    for h in handles:
      h.wait()
    pltpu.sync_copy(gbuf_v.at[pl.ds(0, EPW)], out_hbm.at[pl.ds(ebase, EPW)])

  return gather_kernel(src, dst, mflat)


def _sum_body(cnt_ref, du_ref, dp_ref):
  i = pl.program_id(0)
  blk = cnt_ref[...]
  du_ref[...] = jnp.sum(blk, axis=1, keepdims=True)

  @pl.when(i == 0)
  def _init():
    dp_ref[...] = jnp.zeros_like(dp_ref)

  dp_ref[...] += jnp.sum(blk, axis=0, keepdims=True)


def _deg_sums(cnt):
  return pl.pallas_call(
      _sum_body,
      grid=(25,),
      in_specs=[pl.BlockSpec((200, NPAD), lambda i: (i, 0))],
      out_specs=[
          pl.BlockSpec((200, 1), lambda i: (i, 0)),
          pl.BlockSpec((1, NPAD), lambda i: (0, 0)),
      ],
      out_shape=[
          jax.ShapeDtypeStruct((NU, 1), jnp.float32),
          jax.ShapeDtypeStruct((1, NPAD), jnp.float32),
      ],
  )(cnt)


def _embed_body(f_ref, w_ref, b_ref, x_ref):
  x_ref[...] = (
      jnp.dot(f_ref[...], w_ref[0], preferred_element_type=jnp.float32)
      + b_ref[0])


def _embed(feats, wst, bst):
  return pl.pallas_call(
      _embed_body,
      grid=(10,),
      in_specs=[
          pl.BlockSpec((1000, 128), lambda i: (i, 0)),
          pl.BlockSpec((1, 128, EMB), lambda i: (i // 5, 0, 0)),
          pl.BlockSpec((1, 1, EMB), lambda i: (i // 5, 0, 0)),
      ],
      out_specs=pl.BlockSpec((1000, EMB), lambda i: (i, 0)),
      out_shape=jax.ShapeDtypeStruct((NN, EMB), jnp.float32),
  )(feats, wst, bst)


def _y_body(x_ref, w_ref, deg_ref, y_ref, dinv_ref):
  dinv = lax.rsqrt(deg_ref[...] + 1.0)
  y_ref[...] = dinv * jnp.dot(
      x_ref[...], w_ref[...], preferred_element_type=jnp.float32)
  dinv_ref[...] = dinv


def _y_scaled(x, w, deg):
  return pl.pallas_call(
      _y_body,
      grid=(10,),
      in_specs=[
          pl.BlockSpec((1000, EMB), lambda i: (i, 0)),
          pl.BlockSpec((EMB, EMB), lambda i: (0, 0)),
          pl.BlockSpec((1000, 1), lambda i: (i, 0)),
      ],
      out_specs=[
          pl.BlockSpec((1000, EMB), lambda i: (i, 0)),
          pl.BlockSpec((1000, 1), lambda i: (i, 0)),
      ],
      out_shape=[
          jax.ShapeDtypeStruct((NN, EMB), jnp.float32),
          jax.ShapeDtypeStruct((NN, 1), jnp.float32),
      ],
  )(x, w, deg)


def _agg_body(cnt_ref, yrhs_ref, yself_ref, xself_ref, dinv_ref, b_ref, g_ref,
              beta_ref, rm_ref, rv_ref, out_ref, acc_ref, *, transp, relu):
  j = pl.program_id(1)

  @pl.when(j == 0)
  def _init():
    acc_ref[...] = jnp.zeros_like(acc_ref)

  if transp:
    acc_ref[...] += lax.dot_general(
        cnt_ref[...], yrhs_ref[...], (((0,), (0,)), ((), ())),
        preferred_element_type=jnp.float32)
  else:
    acc_ref[...] += jnp.dot(
        cnt_ref[...], yrhs_ref[...], preferred_element_type=jnp.float32)

  @pl.when(j == pl.num_programs(1) - 1)
  def _epilogue():
    dinv = dinv_ref[...]
    h = dinv * (acc_ref[...] + yself_ref[...]) + b_ref[...]
    h = (h - rm_ref[...]) * lax.rsqrt(rv_ref[...] + BN_EPS) * g_ref[...] \
        + beta_ref[...]
    if relu:
      h = jnp.maximum(h, 0.0)
    out_ref[...] = h + xself_ref[...]


def _agg(cnt, yrhs, yself, xself, dinv, b, g, beta, rm, rv, *, transp, relu):
  if transp:
    cnt_map = lambda i, j: (j, i)
    mblk, kblk, nrow = 1024, 1000, NPAD
  else:
    cnt_map = lambda i, j: (i, j)
    mblk, kblk, nrow = 1000, 1024, NU
  vec = lambda i, j: (i, 0)
  par = lambda i, j: (0, 0)
  return pl.pallas_call(
      functools.partial(_agg_body, transp=transp, relu=relu),
      grid=(5, 5),
      in_specs=[
          pl.BlockSpec((1000, 1024), cnt_map),
          pl.BlockSpec((kblk, EMB), lambda i, j: (j, 0)),
          pl.BlockSpec((mblk, EMB), vec),
          pl.BlockSpec((mblk, EMB), vec),
          pl.BlockSpec((mblk, 1), vec),
          pl.BlockSpec((1, EMB), par),
          pl.BlockSpec((1, EMB), par),
          pl.BlockSpec((1, EMB), par),
          pl.BlockSpec((1, EMB), par),
          pl.BlockSpec((1, EMB), par),
      ],
      out_specs=pl.BlockSpec((mblk, EMB), vec),
      out_shape=jax.ShapeDtypeStruct((nrow, EMB), jnp.float32),
      scratch_shapes=[pltpu.VMEM((mblk, EMB), jnp.float32)],
  )(cnt, yrhs, yself, xself, dinv, b, g, beta, rm, rv)


def _scale_body(x_ref, sc_ref, d_ref):
  x = x_ref[...]
  n = jnp.sqrt(jnp.sum(x * x, axis=1, keepdims=True))
  d_ref[...] = x * sc_ref[0] / jnp.maximum(n, 1e-12)


def _pred_scale(x2, scale):
  return pl.pallas_call(
      _scale_body,
      grid=(10,),
      in_specs=[
          pl.BlockSpec((1000, EMB), lambda i: (i, 0)),
          pl.BlockSpec((1, 1, EMB), lambda i: (i // 5, 0, 0)),
      ],
      out_specs=pl.BlockSpec((1000, EMB), lambda i: (i, 0)),
      out_shape=jax.ShapeDtypeStruct((NN, EMB), jnp.float32),
  )(x2, scale)


def _m_body(a_ref, b_ref, bp_ref, m_ref):
  m_ref[...] = lax.dot_general(
      a_ref[...], b_ref[...], (((1,), (1,)), ((), ())),
      preferred_element_type=jnp.float32) + bp_ref[0, 0]


def _pred_matrix(a, b, bpred):
  return pl.pallas_call(
      _m_body,
      grid=(5, 5),
      in_specs=[
          pl.BlockSpec((1000, EMB), lambda i, j: (i, 0)),
          pl.BlockSpec((1024, EMB), lambda i, j: (j, 0)),
          pl.BlockSpec((1, 1), lambda i, j: (0, 0)),
      ],
      out_specs=pl.BlockSpec((1000, 1024), lambda i, j: (i, j)),
      out_shape=jax.ShapeDtypeStruct((NU, NPAD), jnp.float32),
  )(a, b, bpred)


def kernel(edge_index, user_features, product_features, Wu, bu, Wp, bp, W1, b1,
           g1, beta1, rm1, rv1, W2, b2, g2, beta2, rm2, rv2, Wpred, bpred):
  src = edge_index[0]
  dst = edge_index[1]

  cnt_flat = _cnt_sc(src, dst)
  cnt = cnt_flat.reshape(NU, NPAD)

  du, dp = _deg_sums(cnt)
  deg = jnp.concatenate([du, dp.T[:NP]], axis=0)

  feats = jnp.concatenate([user_features, product_features], axis=0)
  wst = jnp.stack([Wu, Wp])
  bst = jnp.stack([bu, bp]).reshape(2, 1, EMB)
  x = _embed(feats, wst, bst)

  b1r = b1.reshape(1, EMB)
  g1r, beta1r = g1.reshape(1, EMB), beta1.reshape(1, EMB)
  rm1r, rv1r = rm1.reshape(1, EMB), rv1.reshape(1, EMB)
  b2r = b2.reshape(1, EMB)
  g2r, beta2r = g2.reshape(1, EMB), beta2.reshape(1, EMB)
  rm2r, rv2r = rm2.reshape(1, EMB), rv2.reshape(1, EMB)

  zpad = jnp.zeros((NPAD - NP, EMB), jnp.float32)
  zpad1 = jnp.zeros((NPAD - NP, 1), jnp.float32)

  def pad_p(arr):
    return jnp.concatenate([arr, zpad if arr.shape[1] == EMB else zpad1],
                           axis=0)

  y1, dinv = _y_scaled(x, W1, deg)
  dinv_pp = pad_p(dinv[NU:])
  x1u = _agg(cnt, pad_p(y1[NU:]), y1[:NU], x[:NU], dinv[:NU], b1r, g1r,
             beta1r, rm1r, rv1r, transp=False, relu=True)
  x1p = _agg(cnt, y1[:NU], pad_p(y1[NU:]), pad_p(x[NU:]), dinv_pp, b1r, g1r,
             beta1r, rm1r, rv1r, transp=True, relu=True)[:NP]
  x1 = jnp.concatenate([x1u, x1p], axis=0)

  y2, _ = _y_scaled(x1, W2, deg)
  x2u = _agg(cnt, pad_p(y2[NU:]), y2[:NU], x1[:NU], dinv[:NU], b2r, g2r,
             beta2r, rm2r, rv2r, transp=False, relu=False)
  x2p = _agg(cnt, y2[:NU], pad_p(y2[NU:]), pad_p(x1[NU:]), dinv_pp, b2r, g2r,
             beta2r, rm2r, rv2r, transp=True, relu=False)[:NP]

  x2 = jnp.concatenate([x2u, x2p], axis=0)
  scale = jnp.stack([Wpred[:, 0],
                     jnp.ones((EMB,), jnp.float32)]).reshape(2, 1, EMB)
  d = _pred_scale(x2, scale)
  m = _pred_matrix(d[:NU], pad_p(d[NU:]), bpred.reshape(1, 1))

  return _gather_sc(src, dst, m.reshape(-1))


# speedup vs baseline: 25.6684x; 1.0060x over previous
import functools

import jax
import jax.numpy as jnp
from jax import lax
from jax.experimental import pallas as pl
from jax.experimental.pallas import tpu as pltpu
from jax.experimental.pallas import tpu_sc as plsc

NU = 5000
NP = 5000
NPAD = 5120
NN = NU + NP
EMB = 256
E = 320000
NCELL = NU * NPAD
BN_EPS = 1e-5

R_RANGE = 1_600_000
DUMP = 2048
ACCN = R_RANGE + DUMP
NSWEEP = NCELL // (2 * R_RANGE)
EPT = E // 16
CCH = 2048
NCH = -(-EPT // CCH)
CLAST = EPT - (NCH - 1) * CCH
CROWS = CCH // 128
STRIPE = R_RANGE // 16
DCH = 4000
DRN = 10000

EPW = E // 32
NROW_C = -(-EPW // 128)
EPW_PAD = NROW_C * 128


def _cnt_sc(src, dst):
  mesh = plsc.VectorSubcoreMesh(core_axis_name="c", subcore_axis_name="s")

  @functools.partial(
      pl.kernel,
      out_type=jax.ShapeDtypeStruct((NCELL,), jnp.float32),
      mesh=mesh,
      scratch_types=[
          pltpu.VMEM((CCH,), jnp.int32),
          pltpu.VMEM((CCH,), jnp.int32),
          pltpu.VMEM((CCH,), jnp.int32),
          pltpu.VMEM((CCH,), jnp.float32),
          pltpu.VMEM((DCH,), jnp.float32),
          pltpu.VMEM((DRN,), jnp.float32),
          pltpu.VMEM_SHARED((ACCN,), jnp.float32),
          pltpu.SemaphoreType.DMA,
      ],
  )
  def cnt_kernel(src_hbm, dst_hbm, cnt_hbm, src_v, dst_v, idx_v, ones_v,
                 zero_v, drain_v, acc_sh, sem):
    cid = lax.axis_index("c")
    sid = lax.axis_index("s")
    ebase = sid * EPT

    def fill_ones(i, carry):
      ones_v[pl.ds(i * 16, 16)] = jnp.full((16,), 1.0, jnp.float32)
      return carry

    lax.fori_loop(0, CCH // 16, fill_ones, 0)

    def fill_zero(i, carry):
      zero_v[pl.ds(i * 16, 16)] = jnp.full((16,), 0.0, jnp.float32)
      return carry

    lax.fori_loop(0, DCH // 16, fill_zero, 0)

    for sweep in range(NSWEEP):
      lo = (2 * sweep + cid) * R_RANGE

      def zero_stripe(k, carry):
        pltpu.sync_copy(zero_v, acc_sh.at[pl.ds(sid * STRIPE + k * DCH, DCH)])
        return carry

      lax.fori_loop(0, STRIPE // DCH, zero_stripe, 0)
      plsc.subcore_barrier()

      def do_chunk(c, carry):
        nload = jnp.where(c == NCH - 1, CLAST, CCH)

        @pl.when(c < NCH - 1)
        def _ld_full():
          pltpu.sync_copy(src_hbm.at[pl.ds(ebase + c * CCH, CCH)], src_v)
          pltpu.sync_copy(dst_hbm.at[pl.ds(ebase + c * CCH, CCH)], dst_v)

        @pl.when(c == NCH - 1)
        def _ld_tail():
          pltpu.sync_copy(src_hbm.at[pl.ds(ebase + c * CCH, CLAST)],
                          src_v.at[pl.ds(0, CLAST)])
          pltpu.sync_copy(dst_hbm.at[pl.ds(ebase + c * CCH, CLAST)],
                          dst_v.at[pl.ds(0, CLAST)])

        def build_idx(i, carry2):
          s16 = src_v[pl.ds(i * 16, 16)]
          d16 = dst_v[pl.ds(i * 16, 16)]
          flat = s16 * NPAD + d16 - NU
          local = flat - lo
          lane = i * 16 + lax.iota(jnp.int32, 16)
          ok = (lane < nload) & (local >= 0) & (local < R_RANGE)
          spill = R_RANGE + (flat & (DUMP - 1))
          idx_v[pl.ds(i * 16, 16)] = jnp.where(ok, local, spill)
          return carry2

        lax.fori_loop(0, CCH // 16, build_idx, 0)

        pltpu.async_copy(ones_v, acc_sh.at[idx_v], sem, add=True).wait()
        return carry

      lax.fori_loop(0, NCH, do_chunk, 0)
      plsc.subcore_barrier()

      def drain_chunk(k, carry):
        off = sid * STRIPE + k * DRN
        pltpu.sync_copy(acc_sh.at[pl.ds(off, DRN)], drain_v)
        pltpu.sync_copy(drain_v, cnt_hbm.at[pl.ds(lo + off, DRN)])
        return carry

      lax.fori_loop(0, STRIPE // DRN, drain_chunk, 0)

  return cnt_kernel(src, dst)


def _gather_sc(src, dst, mflat):
  mesh = plsc.VectorSubcoreMesh(core_axis_name="c", subcore_axis_name="s")

  @functools.partial(
      pl.kernel,
      out_type=jax.ShapeDtypeStruct((E,), jnp.float32),
      mesh=mesh,
      scratch_types=[
          pltpu.VMEM((EPW_PAD,), jnp.int32),
          pltpu.VMEM((EPW_PAD,), jnp.int32),
          pltpu.VMEM((NROW_C, 128), jnp.int32),
          pltpu.VMEM((EPW_PAD,), jnp.float32),
          pltpu.SemaphoreType.DMA,
      ],
  )
  def gather_kernel(src_hbm, dst_hbm, m_hbm, out_hbm, src_v, dst_v, idx_v,
                    gbuf_v, sem):
    cid = lax.axis_index("c")
    sid = lax.axis_index("s")
    w = sid * 2 + cid
    ebase = w * EPW
    pltpu.sync_copy(src_hbm.at[pl.ds(ebase, EPW)], src_v.at[pl.ds(0, EPW)])
    pltpu.sync_copy(dst_hbm.at[pl.ds(ebase, EPW)], dst_v.at[pl.ds(0, EPW)])

    def fill_pad(i, carry):
      src_v[pl.ds(EPW + i * 16, 16)] = jnp.zeros((16,), jnp.int32)
      dst_v[pl.ds(EPW + i * 16, 16)] = NU + lax.iota(jnp.int32, 16)
      return carry

    lax.fori_loop(0, (EPW_PAD - EPW) // 16, fill_pad, 0)

    def build_idx(i, carry):
      s16 = src_v[pl.ds(i * 16, 16)]
      d16 = dst_v[pl.ds(i * 16, 16)]
      idx_v[i // 8, pl.ds((i % 8) * 16, 16)] = s16 * NPAD + d16 - NU
      return carry

    lax.fori_loop(0, NROW_C * 8, build_idx, 0)

    handles = [
        pltpu.async_copy(m_hbm.at[idx_v.at[j]],
                         gbuf_v.at[pl.ds(j * 128, 128)], sem)
        for j in range(NROW_C)
    ]
    for h in handles:
      h.wait()
    pltpu.sync_copy(gbuf_v.at[pl.ds(0, EPW)], out_hbm.at[pl.ds(ebase, EPW)])

  return gather_kernel(src, dst, mflat)


def _sum_body(cnt_ref, du_ref, dp_ref):
  i = pl.program_id(0)
  blk = cnt_ref[...]
  du_ref[...] = jnp.sum(blk, axis=1, keepdims=True)

  @pl.when(i == 0)
  def _init():
    dp_ref[...] = jnp.zeros_like(dp_ref)

  dp_ref[...] += jnp.sum(blk, axis=0, keepdims=True)


def _deg_sums(cnt):
  return pl.pallas_call(
      _sum_body,
      grid=(25,),
      in_specs=[pl.BlockSpec((200, NPAD), lambda i: (i, 0))],
      out_specs=[
          pl.BlockSpec((200, 1), lambda i: (i, 0)),
          pl.BlockSpec((1, NPAD), lambda i: (0, 0)),
      ],
      out_shape=[
          jax.ShapeDtypeStruct((NU, 1), jnp.float32),
          jax.ShapeDtypeStruct((1, NPAD), jnp.float32),
      ],
  )(cnt)


def _embed_body(f_ref, w_ref, b_ref, x_ref):
  x_ref[...] = (
      jnp.dot(f_ref[...], w_ref[0], preferred_element_type=jnp.float32)
      + b_ref[0])


def _embed(feats, wst, bst):
  return pl.pallas_call(
      _embed_body,
      grid=(10,),
      in_specs=[
          pl.BlockSpec((1000, 128), lambda i: (i, 0)),
          pl.BlockSpec((1, 128, EMB), lambda i: (i // 5, 0, 0)),
          pl.BlockSpec((1, 1, EMB), lambda i: (i // 5, 0, 0)),
      ],
      out_specs=pl.BlockSpec((1000, EMB), lambda i: (i, 0)),
      out_shape=jax.ShapeDtypeStruct((NN, EMB), jnp.float32),
  )(feats, wst, bst)


def _y_body(x_ref, w_ref, deg_ref, y_ref, dinv_ref):
  dinv = lax.rsqrt(deg_ref[...] + 1.0)
  y_ref[...] = dinv * jnp.dot(
      x_ref[...], w_ref[...], preferred_element_type=jnp.float32)
  dinv_ref[...] = dinv


def _y_scaled(x, w, deg):
  return pl.pallas_call(
      _y_body,
      grid=(10,),
      in_specs=[
          pl.BlockSpec((1000, EMB), lambda i: (i, 0)),
          pl.BlockSpec((EMB, EMB), lambda i: (0, 0)),
          pl.BlockSpec((1000, 1), lambda i: (i, 0)),
      ],
      out_specs=[
          pl.BlockSpec((1000, EMB), lambda i: (i, 0)),
          pl.BlockSpec((1000, 1), lambda i: (i, 0)),
      ],
      out_shape=[
          jax.ShapeDtypeStruct((NN, EMB), jnp.float32),
          jax.ShapeDtypeStruct((NN, 1), jnp.float32),
      ],
  )(x, w, deg)


def _agg_body(cnt_ref, yrhs_ref, yself_ref, xself_ref, dinv_ref, b_ref, g_ref,
              beta_ref, rm_ref, rv_ref, out_ref, acc_ref, *, transp, relu):
  j = pl.program_id(1)

  @pl.when(j == 0)
  def _init():
    acc_ref[...] = jnp.zeros_like(acc_ref)

  if transp:
    acc_ref[...] += lax.dot_general(
        cnt_ref[...], yrhs_ref[...], (((0,), (0,)), ((), ())),
        preferred_element_type=jnp.float32)
  else:
    acc_ref[...] += jnp.dot(
        cnt_ref[...], yrhs_ref[...], preferred_element_type=jnp.float32)

  @pl.when(j == pl.num_programs(1) - 1)
  def _epilogue():
    dinv = dinv_ref[...]
    h = dinv * (acc_ref[...] + yself_ref[...]) + b_ref[...]
    h = (h - rm_ref[...]) * lax.rsqrt(rv_ref[...] + BN_EPS) * g_ref[...] \
        + beta_ref[...]
    if relu:
      h = jnp.maximum(h, 0.0)
    out_ref[...] = h + xself_ref[...]


def _agg(cnt, yrhs, yself, xself, dinv, b, g, beta, rm, rv, *, transp, relu):
  if transp:
    cnt_map = lambda i, j: (j, i)
    mblk, kblk, nrow = 1024, 1000, NPAD
  else:
    cnt_map = lambda i, j: (i, j)
    mblk, kblk, nrow = 1000, 1024, NU
  vec = lambda i, j: (i, 0)
  par = lambda i, j: (0, 0)
  return pl.pallas_call(
      functools.partial(_agg_body, transp=transp, relu=relu),
      grid=(5, 5),
      in_specs=[
          pl.BlockSpec((1000, 1024), cnt_map),
          pl.BlockSpec((kblk, EMB), lambda i, j: (j, 0)),
          pl.BlockSpec((mblk, EMB), vec),
          pl.BlockSpec((mblk, EMB), vec),
          pl.BlockSpec((mblk, 1), vec),
          pl.BlockSpec((1, EMB), par),
          pl.BlockSpec((1, EMB), par),
          pl.BlockSpec((1, EMB), par),
          pl.BlockSpec((1, EMB), par),
          pl.BlockSpec((1, EMB), par),
      ],
      out_specs=pl.BlockSpec((mblk, EMB), vec),
      out_shape=jax.ShapeDtypeStruct((nrow, EMB), jnp.float32),
      scratch_shapes=[pltpu.VMEM((mblk, EMB), jnp.float32)],
  )(cnt, yrhs, yself, xself, dinv, b, g, beta, rm, rv)


def _scale_body(x_ref, sc_ref, d_ref):
  x = x_ref[...]
  n = jnp.sqrt(jnp.sum(x * x, axis=1, keepdims=True))
  d_ref[...] = x * sc_ref[0] / jnp.maximum(n, 1e-12)


def _pred_scale(x2, scale):
  return pl.pallas_call(
      _scale_body,
      grid=(10,),
      in_specs=[
          pl.BlockSpec((1000, EMB), lambda i: (i, 0)),
          pl.BlockSpec((1, 1, EMB), lambda i: (i // 5, 0, 0)),
      ],
      out_specs=pl.BlockSpec((1000, EMB), lambda i: (i, 0)),
      out_shape=jax.ShapeDtypeStruct((NN, EMB), jnp.float32),
  )(x2, scale)


def _m_body(a_ref, b_ref, bp_ref, m_ref):
  m_ref[...] = lax.dot_general(
      a_ref[...], b_ref[...], (((1,), (1,)), ((), ())),
      preferred_element_type=jnp.float32) + bp_ref[0, 0]


def _pred_matrix(a, b, bpred):
  return pl.pallas_call(
      _m_body,
      grid=(5, 5),
      in_specs=[
          pl.BlockSpec((1000, EMB), lambda i, j: (i, 0)),
          pl.BlockSpec((1024, EMB), lambda i, j: (j, 0)),
          pl.BlockSpec((1, 1), lambda i, j: (0, 0)),
      ],
      out_specs=pl.BlockSpec((1000, 1024), lambda i, j: (i, j)),
      out_shape=jax.ShapeDtypeStruct((NU, NPAD), jnp.float32),
  )(a, b, bpred)


def kernel(edge_index, user_features, product_features, Wu, bu, Wp, bp, W1, b1,
           g1, beta1, rm1, rv1, W2, b2, g2, beta2, rm2, rv2, Wpred, bpred):
  src = edge_index[0]
  dst = edge_index[1]

  cnt_flat = _cnt_sc(src, dst)
  cnt = cnt_flat.reshape(NU, NPAD)

  du, dp = _deg_sums(cnt)
  deg = jnp.concatenate([du, dp.T[:NP]], axis=0)

  feats = jnp.concatenate([user_features, product_features], axis=0)
  wst = jnp.stack([Wu, Wp])
  bst = jnp.stack([bu, bp]).reshape(2, 1, EMB)
  x = _embed(feats, wst, bst)

  b1r = b1.reshape(1, EMB)
  g1r, beta1r = g1.reshape(1, EMB), beta1.reshape(1, EMB)
  rm1r, rv1r = rm1.reshape(1, EMB), rv1.reshape(1, EMB)
  b2r = b2.reshape(1, EMB)
  g2r, beta2r = g2.reshape(1, EMB), beta2.reshape(1, EMB)
  rm2r, rv2r = rm2.reshape(1, EMB), rv2.reshape(1, EMB)

  zpad = jnp.zeros((NPAD - NP, EMB), jnp.float32)
  zpad1 = jnp.zeros((NPAD - NP, 1), jnp.float32)

  def pad_p(arr):
    return jnp.concatenate([arr, zpad if arr.shape[1] == EMB else zpad1],
                           axis=0)

  y1, dinv = _y_scaled(x, W1, deg)
  dinv_pp = pad_p(dinv[NU:])
  x1u = _agg(cnt, pad_p(y1[NU:]), y1[:NU], x[:NU], dinv[:NU], b1r, g1r,
             beta1r, rm1r, rv1r, transp=False, relu=True)
  x1p = _agg(cnt, y1[:NU], pad_p(y1[NU:]), pad_p(x[NU:]), dinv_pp, b1r, g1r,
             beta1r, rm1r, rv1r, transp=True, relu=True)[:NP]
  x1 = jnp.concatenate([x1u, x1p], axis=0)

  y2, _ = _y_scaled(x1, W2, deg)
  x2u = _agg(cnt, pad_p(y2[NU:]), y2[:NU], x1[:NU], dinv[:NU], b2r, g2r,
             beta2r, rm2r, rv2r, transp=False, relu=False)
  x2p = _agg(cnt, y2[:NU], pad_p(y2[NU:]), pad_p(x1[NU:]), dinv_pp, b2r, g2r,
             beta2r, rm2r, rv2r, transp=True, relu=False)[:NP]

  x2 = jnp.concatenate([x2u, x2p], axis=0)
  scale = jnp.stack([Wpred[:, 0],
                     jnp.ones((EMB,), jnp.float32)]).reshape(2, 1, EMB)
  d = _pred_scale(x2, scale)
  m = _pred_matrix(d[:NU], pad_p(d[NU:]), bpred.reshape(1, 1))

  return _gather_sc(src, dst, m.reshape(-1))
